# Initial kernel scaffold; baseline (speedup 1.0000x reference)
import jax
import jax.numpy as jnp
from jax.experimental import pallas as pl


def kernel(x, edge_index, batch, W_local, b_local, W_global, b_global, W_fus, b_fus, W1, b1, W2, b2):
    raise NotImplementedError("write your pallas kernel here")


# speedup vs baseline: 22.5110x; 22.5110x over previous
import functools

import jax
import jax.numpy as jnp
from jax import lax
from jax.experimental import pallas as pl
from jax.experimental.pallas import tpu as pltpu
from jax.experimental.pallas import tpu_sc as plsc

N = 10000
E = 320000
D = 128
H = 128
G = 64

NC = 2
NS = 16
NW = NC * NS
NP = 10240

CH = 80
NCH = (E // NW) // CH
ROWS_PT = NP // NS

BN = 128
NPB = NP // BN


@functools.lru_cache(maxsize=None)
def _sc_kernels():
    mesh = plsc.VectorSubcoreMesh(core_axis_name="c", subcore_axis_name="s")

    @functools.partial(
        pl.kernel,
        out_type=jax.ShapeDtypeStruct((NC * NP, D), jnp.float32),
        mesh=mesh,
        scratch_types=[
            pltpu.VMEM((NCH, CH), jnp.int32),
            pltpu.VMEM((CH, D), jnp.float32),
            pltpu.VMEM_SHARED((NP, D), jnp.float32),
        ],
        interpret=False,
    )
    def _sc_degree(dst_hbm, zeros_hbm, ones_hbm, out_hbm, idx_v, ones_v,
                   acc_sh):
        c = lax.axis_index("c")
        s = lax.axis_index("s")
        wid = s * NC + c
        base = s * ROWS_PT

        pltpu.sync_copy(ones_hbm, ones_v)
        pltpu.sync_copy(zeros_hbm.at[pl.ds(base, ROWS_PT)],
                        acc_sh.at[pl.ds(base, ROWS_PT)])
        plsc.subcore_barrier()

        pltpu.sync_copy(dst_hbm.at[wid], idx_v)

        @pl.loop(0, NCH)
        def _(i):
            pltpu.sync_copy(ones_v, acc_sh.at[idx_v.at[i]], add=True)

        plsc.subcore_barrier()
        pltpu.sync_copy(acc_sh.at[pl.ds(base, ROWS_PT)],
                        out_hbm.at[pl.ds(c * NP + base, ROWS_PT)])

    @functools.partial(
        pl.kernel,
        out_type=jax.ShapeDtypeStruct((NC * NP, D), jnp.float32),
        mesh=mesh,
        scratch_types=[
            pltpu.VMEM((NCH, CH), jnp.int32),
            pltpu.VMEM((NCH, CH), jnp.int32),
            pltpu.VMEM((CH, D), jnp.float32),
            pltpu.VMEM_SHARED((NP, D), jnp.float32),
            pltpu.SemaphoreType.DMA,
        ],
        interpret=False,
    )
    def _sc_agg(xs_hbm, src_hbm, dst_hbm, zeros_hbm, out_hbm, sidx, didx,
                rows_v, acc_sh, gsem):
        c = lax.axis_index("c")
        s = lax.axis_index("s")
        wid = s * NC + c
        base = s * ROWS_PT

        pltpu.sync_copy(zeros_hbm.at[pl.ds(base, ROWS_PT)],
                        acc_sh.at[pl.ds(base, ROWS_PT)])
        plsc.subcore_barrier()

        pltpu.sync_copy(src_hbm.at[wid], sidx)
        pltpu.sync_copy(dst_hbm.at[wid], didx)

        @pl.loop(0, NCH)
        def _(i):
            pltpu.async_copy(xs_hbm.at[sidx.at[i]], rows_v, gsem).wait()
            pltpu.sync_copy(rows_v, acc_sh.at[didx.at[i]], add=True)

        plsc.subcore_barrier()
        pltpu.sync_copy(acc_sh.at[pl.ds(base, ROWS_PT)],
                        out_hbm.at[pl.ds(c * NP + base, ROWS_PT)])

    return _sc_degree, _sc_agg


def _scale_body(dega_ref, degb_ref, x_ref, o_ref):
    d = dega_ref[:, 0:1] + degb_ref[:, 0:1] + 1.0
    o_ref[...] = x_ref[...] * lax.rsqrt(d)


def _tc_scale(deg_a, deg_b, x_pad):
    return pl.pallas_call(
        _scale_body,
        grid=(NPB,),
        in_specs=[
            pl.BlockSpec((BN, D), lambda i: (i, 0)),
            pl.BlockSpec((BN, D), lambda i: (i, 0)),
            pl.BlockSpec((BN, D), lambda i: (i, 0)),
        ],
        out_specs=pl.BlockSpec((BN, D), lambda i: (i, 0)),
        out_shape=jax.ShapeDtypeStruct((NP, D), jnp.float32),
        interpret=False,
    )(deg_a, deg_b, x_pad)


def _dense_body(dega_ref, degb_ref, ya_ref, yb_ref, xs_ref, b_ref,
                Wl_ref, bl_ref, Wg_ref, bg_ref,
                Wft_ref, Wfb_ref, bf_ref,
                W1_ref, b1_ref, W2_ref, b2_ref,
                o_ref, sums, cnts):
    i = pl.program_id(0)

    @pl.when(i == 0)
    def _():
        sums[...] = jnp.zeros_like(sums)
        cnts[...] = jnp.zeros_like(cnts)

    d = dega_ref[:, 0:1] + degb_ref[:, 0:1] + 1.0
    dinv = lax.rsqrt(d)
    y = (ya_ref[...] + yb_ref[...] + xs_ref[...]) * dinv
    loc = jnp.maximum(
        jnp.dot(y, Wl_ref[...], preferred_element_type=jnp.float32)
        + bl_ref[...], 0.0)
    glo = jnp.maximum(
        jnp.dot(y, Wg_ref[...], preferred_element_type=jnp.float32)
        + bg_ref[...], 0.0)
    fus = jnp.maximum(
        jnp.dot(loc, Wft_ref[...], preferred_element_type=jnp.float32)
        + jnp.dot(glo, Wfb_ref[...], preferred_element_type=jnp.float32)
        + bf_ref[...], 0.0)

    seg = lax.broadcasted_iota(jnp.int32, (BN, G), 1)
    m = (b_ref[...] == seg).astype(jnp.float32)
    dn = (((0,), (0,)), ((), ()))
    sums[...] += lax.dot_general(m, fus, dn, preferred_element_type=jnp.float32)
    cnts[...] += lax.dot_general(m, jnp.ones((BN, D), jnp.float32), dn,
                                 preferred_element_type=jnp.float32)

    @pl.when(i == NPB - 1)
    def _():
        pooled = sums[...] / jnp.maximum(cnts[...], 1.0)
        h = jnp.maximum(
            jnp.dot(pooled, W1_ref[...], preferred_element_type=jnp.float32)
            + b1_ref[...], 0.0)
        o_ref[...] = jnp.dot(h, W2_ref[...], preferred_element_type=jnp.float32) \
            + b2_ref[...]


def _tc_dense(deg_a, deg_b, y_a, y_b, xs, batch2d, Wl, bl, Wg, bg,
              Wft, Wfb, bf, W1, b1, W2p, b2):
    full = lambda shp: pl.BlockSpec(shp, lambda i: tuple(0 for _ in shp))
    return pl.pallas_call(
        _dense_body,
        grid=(NPB,),
        in_specs=[
            pl.BlockSpec((BN, D), lambda i: (i, 0)),
            pl.BlockSpec((BN, D), lambda i: (i, 0)),
            pl.BlockSpec((BN, D), lambda i: (i, 0)),
            pl.BlockSpec((BN, D), lambda i: (i, 0)),
            pl.BlockSpec((BN, D), lambda i: (i, 0)),
            pl.BlockSpec((BN, 1), lambda i: (i, 0)),
            full((D, H)), full((1, H)),
            full((D, H)), full((1, H)),
            full((H, H)), full((H, H)), full((1, H)),
            full((H, H // 2)), full((1, H // 2)),
            full((H // 2, D)), full((1, 1)),
        ],
        out_specs=pl.BlockSpec((G, D), lambda i: (0, 0)),
        out_shape=jax.ShapeDtypeStruct((G, D), jnp.float32),
        scratch_shapes=[
            pltpu.VMEM((G, D), jnp.float32),
            pltpu.VMEM((G, D), jnp.float32),
        ],
        interpret=False,
    )(deg_a, deg_b, y_a, y_b, xs, batch2d, Wl, bl, Wg, bg, Wft, Wfb, bf,
      W1, b1, W2p, b2)


def kernel(x, edge_index, batch, W_local, b_local, W_global, b_global,
           W_fus, b_fus, W1, b1, W2, b2):
    sc_degree, sc_agg = _sc_kernels()
    src3d = edge_index[0].reshape(NW, NCH, CH)
    dst3d = edge_index[1].reshape(NW, NCH, CH)
    x_pad = jnp.concatenate([x, jnp.zeros((NP - N, D), x.dtype)])
    batch2d = jnp.concatenate(
        [batch, jnp.full((NP - N,), jnp.int32(2 * G), batch.dtype)]).reshape(NP, 1)
    zeros_np = jnp.zeros((NP, D), jnp.float32)
    ones_ch = jnp.ones((CH, D), jnp.float32)

    deg2d = sc_degree(dst3d, zeros_np, ones_ch)
    deg_a, deg_b = deg2d[:NP], deg2d[NP:]
    xs = _tc_scale(deg_a, deg_b, x_pad)
    y2d = sc_agg(xs, src3d, dst3d, zeros_np)
    y_a, y_b = y2d[:NP], y2d[NP:]

    W2p = jnp.concatenate([W2, jnp.zeros((H // 2, D - 1), W2.dtype)], axis=1)
    out128 = _tc_dense(
        deg_a, deg_b, y_a, y_b, xs, batch2d,
        W_local, b_local.reshape(1, H),
        W_global, b_global.reshape(1, H),
        W_fus[:H], W_fus[H:], b_fus.reshape(1, H),
        W1, b1.reshape(1, H // 2), W2p, b2.reshape(1, 1))
    return out128[:, :1]
